# baseline (speedup 1.0000x reference)
import functools

import jax
import jax.numpy as jnp
from jax import lax
from jax.experimental import pallas as pl
from jax.experimental.pallas import tpu as pltpu
from jax.experimental.pallas import tpu_sc as plsc

B = 16384
D = 64
L = 16

_info = plsc.get_sparse_core_info()
NC = _info.num_cores
NS = _info.num_subcores
NW = NC * NS
BPW = B // NW
NG = BPW // L

_mesh = plsc.VectorSubcoreMesh(core_axis_name="c", subcore_axis_name="s")


@functools.partial(
    pl.kernel,
    mesh=_mesh,
    out_type=jax.ShapeDtypeStruct((B,), jnp.float32),
    compiler_params=pltpu.CompilerParams(
        needs_layout_passes=False, use_tc_tiling_on_sc=False),
    scratch_types=[
        pltpu.VMEM((BPW,), jnp.int32),
        pltpu.VMEM((BPW,), jnp.int32),
        pltpu.VMEM((BPW,), jnp.int32),
        pltpu.VMEM((BPW, D), jnp.float32),
        pltpu.VMEM((BPW, D), jnp.float32),
        pltpu.VMEM((BPW, D), jnp.float32),
        pltpu.VMEM((BPW,), jnp.float32),
        pltpu.SemaphoreType.DMA,
        pltpu.SemaphoreType.DMA,
        pltpu.SemaphoreType.DMA,
    ],
)
def _transe_sc(heads_hbm, rels_hbm, tails_hbm, ent_hbm, rel_hbm, out_hbm,
               hidx, ridx, tidx, hrows, rrows, trows, outv,
               sem_h, sem_r, sem_t):
    wid = lax.axis_index("s") * NC + lax.axis_index("c")
    base = wid * BPW

    pltpu.sync_copy(heads_hbm.at[pl.ds(base, BPW)], hidx)
    pltpu.sync_copy(rels_hbm.at[pl.ds(base, BPW)], ridx)
    pltpu.sync_copy(tails_hbm.at[pl.ds(base, BPW)], tidx)

    ch = pltpu.async_copy(ent_hbm.at[hidx], hrows, sem_h)
    cr = pltpu.async_copy(rel_hbm.at[ridx], rrows, sem_r)
    ct = pltpu.async_copy(ent_hbm.at[tidx], trows, sem_t)
    ch.wait()
    cr.wait()
    ct.wait()

    lanes = lax.iota(jnp.int32, L)

    def group_body(g, carry):
        acc = jnp.zeros((L,), jnp.float32)
        for j in range(L):
            b = g * L + j
            p = jnp.zeros((L,), jnp.float32)
            for k in range(D // L):
                sl = pl.ds(k * L, L)
                p = p + jnp.abs(hrows[b, sl] + rrows[b, sl] - trows[b, sl])
            acc = jnp.where(lanes == j, jnp.sum(p), acc)
        outv[pl.ds(g * L, L)] = acc
        return carry

    lax.fori_loop(0, NG, group_body, 0)

    pltpu.sync_copy(outv, out_hbm.at[pl.ds(base, BPW)])


def kernel(heads, relations, tails, entity_table, relation_table):
    return _transe_sc(heads, relations, tails, entity_table, relation_table)
